# Initial kernel scaffold; baseline (speedup 1.0000x reference)
import jax
import jax.numpy as jnp
from jax.experimental import pallas as pl


def kernel(x, adj_t, W1, b1, g1, be1, rm1, rv1, W2, b2, g2, be2, rm2, rv2, W3, b3):
    raise NotImplementedError("write your pallas kernel here")


# speedup vs baseline: 17.3429x; 17.3429x over previous
import functools

import jax
import jax.numpy as jnp
from jax import lax
from jax.experimental import pallas as pl
from jax.experimental.pallas import tpu as pltpu
from jax.experimental.pallas import tpu_sc as plsc

N_PAD = 10240
N_EDGES = 320000
NC, NS = 2, 16
CHUNK = 80
EDGES_PER_TILE = N_EDGES // (NC * NS)
ITERS = EDGES_PER_TILE // CHUNK
STRIPE = N_PAD // NS

_MESH = plsc.VectorSubcoreMesh(core_axis_name="c", subcore_axis_name="s")


def _zero_vmem(ref, rows, cols):
    zz = jnp.zeros((16,), jnp.float32)

    def body(t, _):
        i = t // (cols // 16)
        j = t % (cols // 16)
        ref[i, pl.ds(j * 16, 16)] = zz
        return 0

    lax.fori_loop(0, rows * (cols // 16), body, 0)


def _sc_degree(dst):
    @functools.partial(
        pl.kernel,
        mesh=_MESH,
        out_type=jax.ShapeDtypeStruct((NC, N_PAD, 16), jnp.float32),
        scratch_types=[
            pltpu.VMEM((CHUNK,), jnp.int32),
            pltpu.VMEM((CHUNK, 16), jnp.float32),
            pltpu.VMEM((CHUNK, 16), jnp.float32),
            pltpu.VMEM_SHARED((N_PAD, 16), jnp.float32),
        ],
        compiler_params=pltpu.CompilerParams(use_tc_tiling_on_sc=False),
    )
    def k(dst_hbm, out_hbm, didx, zrows, orows, acc):
        c = lax.axis_index("c")
        s = lax.axis_index("s")
        _zero_vmem(zrows, CHUNK, 16)
        one = jnp.ones((16,), jnp.float32)

        def fill_ones(i, _):
            orows[i, pl.ds(0, 16)] = one
            return 0

        lax.fori_loop(0, CHUNK, fill_ones, 0)

        def zero_acc(i, _):
            pltpu.sync_copy(zrows, acc.at[pl.ds(s * STRIPE + i * CHUNK, CHUNK)])
            return 0

        lax.fori_loop(0, STRIPE // CHUNK, zero_acc, 0)
        plsc.subcore_barrier()

        def step(k_, _):
            base = (c * NS + s) * EDGES_PER_TILE + k_ * CHUNK
            pltpu.sync_copy(dst_hbm.at[pl.ds(base, CHUNK)], didx)
            pltpu.sync_copy(orows, acc.at[didx], add=True)
            return 0

        lax.fori_loop(0, ITERS, step, 0)
        plsc.subcore_barrier()

        def flush(i, _):
            r = s * STRIPE + i * CHUNK
            pltpu.sync_copy(acc.at[pl.ds(r, CHUNK)], out_hbm.at[c, pl.ds(r, CHUNK)])
            return 0

        lax.fori_loop(0, STRIPE // CHUNK, flush, 0)

    return k(dst)


def _sc_scatter(h2, src, dst, d):
    @functools.partial(
        pl.kernel,
        mesh=_MESH,
        out_type=jax.ShapeDtypeStruct((NC, N_PAD, d), jnp.float32),
        scratch_types=[
            pltpu.VMEM((2, CHUNK), jnp.int32),
            pltpu.VMEM((2, CHUNK), jnp.int32),
            pltpu.VMEM((2, CHUNK, d), jnp.float32),
            pltpu.VMEM_SHARED((N_PAD, d), jnp.float32),
            pltpu.SemaphoreType.DMA,
            pltpu.SemaphoreType.DMA,
        ],
        compiler_params=pltpu.CompilerParams(use_tc_tiling_on_sc=False),
    )
    def k(h2_hbm, src_hbm, dst_hbm, out_hbm, sidx, didx, rows, acc, sem0,
          sem1):
        c = lax.axis_index("c")
        s = lax.axis_index("s")
        sems = (sem0, sem1)
        tile_base = (c * NS + s) * EDGES_PER_TILE

        def zbody(t, _):
            rows[0, t // (d // 16), pl.ds((t % (d // 16)) * 16, 16)] = \
                jnp.zeros((16,), jnp.float32)
            return 0

        lax.fori_loop(0, CHUNK * (d // 16), zbody, 0)

        def zero_acc(i, _):
            pltpu.sync_copy(rows.at[0],
                            acc.at[pl.ds(s * STRIPE + i * CHUNK, CHUNK)])
            return 0

        lax.fori_loop(0, STRIPE // CHUNK, zero_acc, 0)
        plsc.subcore_barrier()

        def start(buf, chunk):
            base = tile_base + chunk * CHUNK
            pltpu.sync_copy(src_hbm.at[pl.ds(base, CHUNK)], sidx.at[buf])
            pltpu.sync_copy(dst_hbm.at[pl.ds(base, CHUNK)], didx.at[buf])
            pltpu.async_copy(h2_hbm.at[sidx.at[buf]], rows.at[buf],
                             sems[buf])

        def drain(buf):
            pltpu.make_async_copy(h2_hbm.at[sidx.at[buf]], rows.at[buf],
                                  sems[buf]).wait()
            pltpu.sync_copy(rows.at[buf], acc.at[didx.at[buf]], add=True)

        start(0, 0)

        def step(t, _):
            start(1, 2 * t + 1)
            drain(0)
            start(0, 2 * t + 2)
            drain(1)
            return 0

        lax.fori_loop(0, (ITERS - 1) // 2, step, 0)
        drain(0)
        plsc.subcore_barrier()

        r = s * STRIPE
        pltpu.sync_copy(acc.at[pl.ds(r, STRIPE)], out_hbm.at[c, pl.ds(r, STRIPE)])

    return k(h2, src, dst)


ROWS_BLK = 1024


def _tc_layer1(x, W1, degp):
    def body(x_ref, w_ref, p0_ref, p1_ref, h2_ref, dinv_ref):
        deg = p0_ref[0, :, 0:1] + p1_ref[0, :, 0:1] + 1.0
        dinv = lax.rsqrt(deg)
        h = jnp.dot(x_ref[...], w_ref[...], preferred_element_type=jnp.float32)
        h2_ref[...] = h * dinv
        dinv_ref[...] = dinv

    g = N_PAD // ROWS_BLK
    return pl.pallas_call(
        body,
        grid=(g,),
        in_specs=[
            pl.BlockSpec((ROWS_BLK, 128), lambda i: (i, 0)),
            pl.BlockSpec((128, 128), lambda i: (0, 0)),
            pl.BlockSpec((1, ROWS_BLK, 16), lambda i: (0, i, 0)),
            pl.BlockSpec((1, ROWS_BLK, 16), lambda i: (1, i, 0)),
        ],
        out_specs=[
            pl.BlockSpec((ROWS_BLK, 128), lambda i: (i, 0)),
            pl.BlockSpec((ROWS_BLK, 1), lambda i: (i, 0)),
        ],
        out_shape=[
            jax.ShapeDtypeStruct((N_PAD, 128), jnp.float32),
            jax.ShapeDtypeStruct((N_PAD, 1), jnp.float32),
        ],
    )(x, W1, degp, degp)


def _tc_mid(P, h2, dinv, b, gam, bet, rm, rv, W, dout):
    din = h2.shape[1]

    def body(p_ref, h2_ref, dinv_ref, b_ref, g_ref, be_ref, rm_ref, rv_ref,
             w_ref, o_ref):
        acc = (p_ref[0] + p_ref[1] + h2_ref[...]) * dinv_ref[...] + b_ref[...]
        y = g_ref[...] * (acc - rm_ref[...]) * lax.rsqrt(rv_ref[...] + 1e-5) \
            + be_ref[...]
        y = jnp.maximum(y, 0.0)
        h = jnp.dot(y, w_ref[...], preferred_element_type=jnp.float32)
        o_ref[...] = h * dinv_ref[...]

    g = N_PAD // ROWS_BLK
    vec = lambda: pl.BlockSpec((1, din), lambda i: (0, 0))
    return pl.pallas_call(
        body,
        grid=(g,),
        in_specs=[
            pl.BlockSpec((2, ROWS_BLK, din), lambda i: (0, i, 0)),
            pl.BlockSpec((ROWS_BLK, din), lambda i: (i, 0)),
            pl.BlockSpec((ROWS_BLK, 1), lambda i: (i, 0)),
            vec(), vec(), vec(), vec(), vec(),
            pl.BlockSpec((din, dout), lambda i: (0, 0)),
        ],
        out_specs=pl.BlockSpec((ROWS_BLK, dout), lambda i: (i, 0)),
        out_shape=jax.ShapeDtypeStruct((N_PAD, dout), jnp.float32),
    )(P, h2, dinv, b.reshape(1, -1), gam.reshape(1, -1), bet.reshape(1, -1),
      rm.reshape(1, -1), rv.reshape(1, -1), W)


def _tc_final(P, h3, dinv, b, dout):
    def body(p_ref, h_ref, dinv_ref, b_ref, o_ref):
        o_ref[...] = (p_ref[0] + p_ref[1] + h_ref[...]) * dinv_ref[...] \
            + b_ref[...]

    g = N_PAD // ROWS_BLK
    return pl.pallas_call(
        body,
        grid=(g,),
        in_specs=[
            pl.BlockSpec((2, ROWS_BLK, dout), lambda i: (0, i, 0)),
            pl.BlockSpec((ROWS_BLK, dout), lambda i: (i, 0)),
            pl.BlockSpec((ROWS_BLK, 1), lambda i: (i, 0)),
            pl.BlockSpec((1, dout), lambda i: (0, 0)),
        ],
        out_specs=pl.BlockSpec((ROWS_BLK, dout), lambda i: (i, 0)),
        out_shape=jax.ShapeDtypeStruct((N_PAD, dout), jnp.float32),
    )(P, h3, dinv, b.reshape(1, -1))


def kernel(x, adj_t, W1, b1, g1, be1, rm1, rv1, W2, b2, g2, be2, rm2, rv2,
           W3, b3):
    src = adj_t[0]
    dst = adj_t[1]
    x_pad = jnp.pad(x, ((0, N_PAD - x.shape[0]), (0, 0)))
    W3p = jnp.pad(W3, ((0, 0), (0, 64 - W3.shape[1])))
    b3p = jnp.pad(b3, (0, 64 - b3.shape[0]))

    degp = _sc_degree(dst)
    h2, dinv = _tc_layer1(x_pad, W1, degp)
    P1 = _sc_scatter(h2, src, dst, 128)
    h2b = _tc_mid(P1, h2, dinv, b1, g1, be1, rm1, rv1, W2, 128)
    P2 = _sc_scatter(h2b, src, dst, 128)
    h3 = _tc_mid(P2, h2b, dinv, b2, g2, be2, rm2, rv2, W3p, 64)
    P3 = _sc_scatter(h3, src, dst, 64)
    out = _tc_final(P3, h3, dinv, b3p, 64)
    return out[:10000, :40]
